# baseline (speedup 1.0000x reference)
import functools

import jax
import jax.numpy as jnp
from jax import lax
from jax.experimental import pallas as pl
from jax.experimental.pallas import tpu as pltpu
from jax.experimental.pallas import tpu_sc as plsc

B, N, C = 8, 2048, 512
K = N // 2

_PIECES = 1
_PB = B // _PIECES


_RANK_CH = 256


def _score_rank_body(x_ref, w_ref, b_ref, s_in_ref, score_ref, rank_ref):
    xb = x_ref[0]
    w = w_ref[0]
    score_ref[0, 0, :] = jnp.sum(xb * w[None, :], axis=1) + b_ref[0]
    s = s_in_ref[0, 0, :]
    iota_n = lax.iota(jnp.int32, N)
    ones = jnp.ones((N, 1), jnp.float32)
    for c in range(N // _RANK_CH):
        sc = s[c * _RANK_CH:(c + 1) * _RANK_CH]
        ic = iota_n[c * _RANK_CH:(c + 1) * _RANK_CH]
        ge = (s[None, :] >= sc[:, None])
        gt = (s[None, :] > sc[:, None])
        jl = (iota_n[None, :] < ic[:, None])
        cmp = jnp.where(jl, ge.astype(jnp.float32),
                        gt.astype(jnp.float32))
        cnt = jax.lax.dot(cmp, ones,
                          preferred_element_type=jnp.float32)[:, 0]
        rank_ref[0, 0, c * _RANK_CH:(c + 1) * _RANK_CH] = cnt.astype(jnp.int32)


def _score_rank_piece(x, W, b, s_in, lo):
    return pl.pallas_call(
        _score_rank_body,
        grid=(_PB,),
        in_specs=[
            pl.BlockSpec((1, N, C), lambda i: (i + lo, 0, 0)),
            pl.BlockSpec((1, C), lambda i: (0, 0)),
            pl.BlockSpec((1,), lambda i: (0,)),
            pl.BlockSpec((1, 1, N), lambda i: (i + lo, 0, 0)),
        ],
        out_specs=[
            pl.BlockSpec((1, 1, N), lambda i: (i, 0, 0)),
            pl.BlockSpec((1, 1, N), lambda i: (i, 0, 0)),
        ],
        out_shape=[
            jax.ShapeDtypeStruct((_PB, 1, N), jnp.float32),
            jax.ShapeDtypeStruct((_PB, 1, N), jnp.int32),
        ],
    )(x, W, b, s_in)


_NW = 32
_XCH = 32
_ACH = 8


def _make_sc_body(nb, pbase):
    tpb = _NW // nb
    slots = K // tpb

    def _sc_body(rank_hbm, x_hbm, adj_hbm, newx_hbm, newadj_hbm,
                 rank_v, sidx_v, idx_v, xbuf0, xbuf1, abuf0, abuf1,
                 obuf0, obuf1, semx0, semx1, sema0, sema1, semo0, semo1):
        nc = 2
        wid = lax.axis_index("s") * nc + lax.axis_index("c")
        b = wid // tpb
        bg = b + pbase
        q = wid % tpb

        with jax.named_scope("a_sidx"):
            pltpu.sync_copy(rank_hbm.at[b], rank_v)
            iota = lax.iota(jnp.int32, 16)

            @plsc.parallel_loop(0, N // 16, unroll=4)
            def _build(i):
                rv = rank_v[pl.ds(i * 16, 16)]
                m = rv < K
                rvc = jnp.where(m, rv, 0)
                plsc.store_scatter(sidx_v, [rvc], iota + i * 16, mask=m)

        @plsc.parallel_loop(0, slots // 16, unroll=4)
        def _mkidx(i):
            sv = sidx_v[pl.ds(q * slots + i * 16, 16)]
            idx_v[pl.ds(i * 16, 16)] = sv + bg * N

        out_base = b * K + q * slots

        xbufs, xsems = (xbuf0, xbuf1), (semx0, semx1)
        nxc = slots // _XCH

        def xin(ci, par):
            return pltpu.async_copy(
                x_hbm.at[idx_v.at[pl.ds(ci * _XCH, _XCH)]],
                xbufs[par], xsems[par])

        with jax.named_scope("b1_xgather"):
            pend = [xin(0, 0), xin(1, 1)]
            for ci in range(nxc):
                par = ci % 2
                pend[par].wait()
                pltpu.sync_copy(xbufs[par],
                                newx_hbm.at[pl.ds(out_base + ci * _XCH, _XCH)])
                if ci + 2 < nxc:
                    pend[par] = xin(ci + 2, par)

        abufs, asems = (abuf0, abuf1), (sema0, sema1)
        obufs, osems = (obuf0, obuf1), (semo0, semo1)
        nac = slots // _ACH

        def ain_src(ci):
            return adj_hbm.at[idx_v.at[pl.ds(ci * _ACH, _ACH)]]

        def aout_dst(ci):
            return newadj_hbm.at[pl.ds(out_base + ci * _ACH, _ACH)]

        def colgather(buf, ob):
            @plsc.parallel_loop(0, K // 16, unroll=2)
            def _cols(cc):
                cidx = sidx_v[pl.ds(cc * 16, 16)]
                for j in range(_ACH):
                    jv = jnp.full((16,), j, jnp.int32)
                    vals = plsc.load_gather(buf, [jv, cidx])
                    ob[j, pl.ds(cc * 16, 16)] = vals

        def astep(ci, first, last):
            par = ci % 2
            pltpu.make_async_copy(ain_src(ci), abufs[par], asems[par]).wait()
            if not first:
                pltpu.make_async_copy(obufs[par], aout_dst(ci - 2),
                                      osems[par]).wait()
            colgather(abufs[par], obufs[par])
            pltpu.async_copy(obufs[par], aout_dst(ci), osems[par])
            if not last:
                pltpu.async_copy(ain_src(ci + 2), abufs[par], asems[par])

        with jax.named_scope("b2_head"):
            pltpu.async_copy(ain_src(0), abuf0, sema0)
            pltpu.async_copy(ain_src(1), abuf1, sema1)
            astep(0, True, False)
            astep(1, True, False)

        def apair(p, _):
            for par in range(2):
                ci = 2 * p + par
                pltpu.make_async_copy(ain_src(ci), abufs[par],
                                      asems[par]).wait()
                pltpu.make_async_copy(obufs[par], aout_dst(ci - 2),
                                      osems[par]).wait()
                colgather(abufs[par], obufs[par])
                pltpu.async_copy(obufs[par], aout_dst(ci), osems[par])
                pltpu.async_copy(ain_src(ci + 2), abufs[par], asems[par])
            return 0

        with jax.named_scope("b2_main"):
            lax.fori_loop(1, nac // 2 - 1, apair, 0)
        with jax.named_scope("b2_tail"):
            for ci in (nac - 2, nac - 1):
                astep(ci, False, True)
            for par in range(2):
                pltpu.make_async_copy(obufs[par], aout_dst(nac - 2 + par),
                                      osems[par]).wait()

    return _sc_body


def _sc_gather(rank, x2d, adj2d, pbase):
    nb = rank.shape[0]
    mesh = plsc.VectorSubcoreMesh(core_axis_name="c", subcore_axis_name="s")
    return pl.kernel(
        _make_sc_body(nb, pbase),
        out_type=[
            jax.ShapeDtypeStruct((nb * K, C), jnp.float32),
            jax.ShapeDtypeStruct((nb * K, K), jnp.float32),
        ],
        mesh=mesh,
        scratch_types=[
            pltpu.VMEM((N,), jnp.int32),
            pltpu.VMEM((K,), jnp.int32),
            pltpu.VMEM((K // (_NW // nb),), jnp.int32),
            pltpu.VMEM((_XCH, C), jnp.float32),
            pltpu.VMEM((_XCH, C), jnp.float32),
            pltpu.VMEM((_ACH, N), jnp.float32),
            pltpu.VMEM((_ACH, N), jnp.float32),
            pltpu.VMEM((_ACH, K), jnp.float32),
            pltpu.VMEM((_ACH, K), jnp.float32),
            pltpu.SemaphoreType.DMA,
            pltpu.SemaphoreType.DMA,
            pltpu.SemaphoreType.DMA,
            pltpu.SemaphoreType.DMA,
            pltpu.SemaphoreType.DMA,
            pltpu.SemaphoreType.DMA,
        ],
        compiler_params=pltpu.CompilerParams(needs_layout_passes=False),
    )(rank, x2d, adj2d)


def kernel(x, adj, W, b):
    s_parity = (x @ W.T + b)[..., 0]

    x2d = x.reshape(B * N, C)
    adj2d = adj.reshape(B * N, N)
    scores, newxs, newadjs = [], [], []
    for p in range(_PIECES):
        lo = p * _PB
        score3, rank3 = _score_rank_piece(x, W, b,
                                          s_parity.reshape(B, 1, N), lo)
        scores.append(score3.reshape(_PB, N))
        new_x, new_adj = _sc_gather(rank3.reshape(_PB, N), x2d, adj2d, lo)
        newxs.append(new_x.reshape(_PB, K, C))
        newadjs.append(new_adj.reshape(_PB, K, K))

    return (jnp.concatenate(newxs, axis=0),
            jnp.concatenate(newadjs, axis=0),
            jnp.concatenate(scores, axis=0))


# speedup vs baseline: 1.0092x; 1.0092x over previous
import functools

import jax
import jax.numpy as jnp
from jax import lax
from jax.experimental import pallas as pl
from jax.experimental.pallas import tpu as pltpu
from jax.experimental.pallas import tpu_sc as plsc

B, N, C = 8, 2048, 512
K = N // 2

_PIECES = 1
_PB = B // _PIECES


_RANK_CH = 256


def _score_rank_body(x_ref, w_ref, b_ref, s_in_ref, score_ref, rank_ref):
    xb = x_ref[0]
    w = w_ref[0]
    score_ref[0, 0, :] = jnp.sum(xb * w[None, :], axis=1) + b_ref[0]
    s = s_in_ref[0, 0, :]
    iota_n = lax.iota(jnp.int32, N)
    ones = jnp.ones((N, 1), jnp.float32)
    for c in range(N // _RANK_CH):
        sc = s[c * _RANK_CH:(c + 1) * _RANK_CH]
        ic = iota_n[c * _RANK_CH:(c + 1) * _RANK_CH]
        ge = (s[None, :] >= sc[:, None])
        gt = (s[None, :] > sc[:, None])
        jl = (iota_n[None, :] < ic[:, None])
        cmp = jnp.where(jl, ge.astype(jnp.float32),
                        gt.astype(jnp.float32))
        cnt = jax.lax.dot(cmp, ones,
                          preferred_element_type=jnp.float32)[:, 0]
        rank_ref[0, 0, c * _RANK_CH:(c + 1) * _RANK_CH] = cnt.astype(jnp.int32)


def _score_rank_piece(x, W, b, s_in, lo):
    return pl.pallas_call(
        _score_rank_body,
        grid=(_PB,),
        in_specs=[
            pl.BlockSpec((1, N, C), lambda i: (i + lo, 0, 0)),
            pl.BlockSpec((1, C), lambda i: (0, 0)),
            pl.BlockSpec((1,), lambda i: (0,)),
            pl.BlockSpec((1, 1, N), lambda i: (i + lo, 0, 0)),
        ],
        out_specs=[
            pl.BlockSpec((1, 1, N), lambda i: (i, 0, 0)),
            pl.BlockSpec((1, 1, N), lambda i: (i, 0, 0)),
        ],
        out_shape=[
            jax.ShapeDtypeStruct((_PB, 1, N), jnp.float32),
            jax.ShapeDtypeStruct((_PB, 1, N), jnp.int32),
        ],
    )(x, W, b, s_in)


_NW = 32
_XCH = 64
_ACH = 8


def _make_sc_body(nb, pbase):
    tpb = _NW // nb
    slots = K // tpb

    def _sc_body(rank_hbm, x_hbm, adj_hbm, newx_hbm, newadj_hbm,
                 rank_v, sidx_v, idx_v, xbuf0, xbuf1, abuf0, abuf1,
                 obuf0, obuf1, semx0, semx1, sema0, sema1, semo0, semo1):
        nc = 2
        wid = lax.axis_index("s") * nc + lax.axis_index("c")
        b = wid // tpb
        bg = b + pbase
        q = wid % tpb

        with jax.named_scope("a_sidx"):
            pltpu.sync_copy(rank_hbm.at[b], rank_v)
            iota = lax.iota(jnp.int32, 16)

            @plsc.parallel_loop(0, N // 16, unroll=4)
            def _build(i):
                rv = rank_v[pl.ds(i * 16, 16)]
                m = rv < K
                rvc = jnp.where(m, rv, 0)
                plsc.store_scatter(sidx_v, [rvc], iota + i * 16, mask=m)

        @plsc.parallel_loop(0, slots // 16, unroll=4)
        def _mkidx(i):
            sv = sidx_v[pl.ds(q * slots + i * 16, 16)]
            idx_v[pl.ds(i * 16, 16)] = sv + bg * N

        out_base = b * K + q * slots

        xbufs, xsems = (xbuf0, xbuf1), (semx0, semx1)
        nxc = slots // _XCH

        def xin(ci, par):
            return pltpu.async_copy(
                x_hbm.at[idx_v.at[pl.ds(ci * _XCH, _XCH)]],
                xbufs[par], xsems[par])

        pltpu.async_copy(
            adj_hbm.at[idx_v.at[pl.ds(0 * _ACH, _ACH)]], abuf0, sema0)
        pltpu.async_copy(
            adj_hbm.at[idx_v.at[pl.ds(1 * _ACH, _ACH)]], abuf1, sema1)

        with jax.named_scope("b1_xgather"):
            pend = [xin(0, 0), xin(1, 1)]
            for ci in range(nxc):
                par = ci % 2
                pend[par].wait()
                pltpu.sync_copy(xbufs[par],
                                newx_hbm.at[pl.ds(out_base + ci * _XCH, _XCH)])
                if ci + 2 < nxc:
                    pend[par] = xin(ci + 2, par)

        abufs, asems = (abuf0, abuf1), (sema0, sema1)
        obufs, osems = (obuf0, obuf1), (semo0, semo1)
        nac = slots // _ACH

        def ain_src(ci):
            return adj_hbm.at[idx_v.at[pl.ds(ci * _ACH, _ACH)]]

        def aout_dst(ci):
            return newadj_hbm.at[pl.ds(out_base + ci * _ACH, _ACH)]

        def colgather(buf, ob):
            @plsc.parallel_loop(0, K // 16, unroll=4)
            def _cols(cc):
                cidx = sidx_v[pl.ds(cc * 16, 16)]
                for j in range(_ACH):
                    jv = jnp.full((16,), j, jnp.int32)
                    vals = plsc.load_gather(buf, [jv, cidx])
                    ob[j, pl.ds(cc * 16, 16)] = vals

        def astep(ci, first, last):
            par = ci % 2
            pltpu.make_async_copy(ain_src(ci), abufs[par], asems[par]).wait()
            if not first:
                pltpu.make_async_copy(obufs[par], aout_dst(ci - 2),
                                      osems[par]).wait()
            colgather(abufs[par], obufs[par])
            pltpu.async_copy(obufs[par], aout_dst(ci), osems[par])
            if not last:
                pltpu.async_copy(ain_src(ci + 2), abufs[par], asems[par])

        with jax.named_scope("b2_head"):
            astep(0, True, False)
            astep(1, True, False)

        def apair(p, _):
            for par in range(2):
                ci = 2 * p + par
                pltpu.make_async_copy(ain_src(ci), abufs[par],
                                      asems[par]).wait()
                pltpu.make_async_copy(obufs[par], aout_dst(ci - 2),
                                      osems[par]).wait()
                colgather(abufs[par], obufs[par])
                pltpu.async_copy(obufs[par], aout_dst(ci), osems[par])
                pltpu.async_copy(ain_src(ci + 2), abufs[par], asems[par])
            return 0

        with jax.named_scope("b2_main"):
            lax.fori_loop(1, nac // 2 - 1, apair, 0)
        with jax.named_scope("b2_tail"):
            for ci in (nac - 2, nac - 1):
                astep(ci, False, True)
            for par in range(2):
                pltpu.make_async_copy(obufs[par], aout_dst(nac - 2 + par),
                                      osems[par]).wait()

    return _sc_body


def _sc_gather(rank, x2d, adj2d, pbase):
    nb = rank.shape[0]
    mesh = plsc.VectorSubcoreMesh(core_axis_name="c", subcore_axis_name="s")
    return pl.kernel(
        _make_sc_body(nb, pbase),
        out_type=[
            jax.ShapeDtypeStruct((nb * K, C), jnp.float32),
            jax.ShapeDtypeStruct((nb * K, K), jnp.float32),
        ],
        mesh=mesh,
        scratch_types=[
            pltpu.VMEM((N,), jnp.int32),
            pltpu.VMEM((K,), jnp.int32),
            pltpu.VMEM((K // (_NW // nb),), jnp.int32),
            pltpu.VMEM((_XCH, C), jnp.float32),
            pltpu.VMEM((_XCH, C), jnp.float32),
            pltpu.VMEM((_ACH, N), jnp.float32),
            pltpu.VMEM((_ACH, N), jnp.float32),
            pltpu.VMEM((_ACH, K), jnp.float32),
            pltpu.VMEM((_ACH, K), jnp.float32),
            pltpu.SemaphoreType.DMA,
            pltpu.SemaphoreType.DMA,
            pltpu.SemaphoreType.DMA,
            pltpu.SemaphoreType.DMA,
            pltpu.SemaphoreType.DMA,
            pltpu.SemaphoreType.DMA,
        ],
        compiler_params=pltpu.CompilerParams(needs_layout_passes=False),
    )(rank, x2d, adj2d)


def kernel(x, adj, W, b):
    s_parity = (x @ W.T + b)[..., 0]

    x2d = x.reshape(B * N, C)
    adj2d = adj.reshape(B * N, N)
    scores, newxs, newadjs = [], [], []
    for p in range(_PIECES):
        lo = p * _PB
        score3, rank3 = _score_rank_piece(x, W, b,
                                          s_parity.reshape(B, 1, N), lo)
        scores.append(score3.reshape(_PB, N))
        new_x, new_adj = _sc_gather(rank3.reshape(_PB, N), x2d, adj2d, lo)
        newxs.append(new_x.reshape(_PB, K, C))
        newadjs.append(new_adj.reshape(_PB, K, K))

    return (jnp.concatenate(newxs, axis=0),
            jnp.concatenate(newadjs, axis=0),
            jnp.concatenate(scores, axis=0))
